# Initial kernel scaffold; baseline (speedup 1.0000x reference)
import jax
import jax.numpy as jnp
from jax.experimental import pallas as pl


def kernel(h, edge_index, e, snorm_n, W, b, gamma, beta):
    raise NotImplementedError("write your pallas kernel here")


# speedup vs baseline: 1.1782x; 1.1782x over previous
import functools

import jax
import jax.numpy as jnp
from jax import lax
from jax.experimental import pallas as pl
from jax.experimental.pallas import tpu as pltpu
from jax.experimental.pallas import tpu_sc as plsc

_N = 10000
_E = 320000
_D = 128
_NW = 32
_NN = 320
_NPAD = _NW * _NN
_CH = 512
_NCHUNK = _E // _CH
_GB = 16
_FLT_BIG = 3.4e38
_AVG_D_LOG = 3.4965
_BN_EPS = 1e-5


def _sc_agg_body(src_hbm, dst_hbm, h_hbm, osum, omax, omin, odeg,
                 dstbuf, srcbuf, sstage, dstage, idxbuf, msgbuf,
                 accsum, accmax, accmin, degbuf, sem):
    wid = lax.axis_index("s") * 2 + lax.axis_index("c")
    lo = (wid * _NN).astype(jnp.int32)
    iota = lax.iota(jnp.int32, 16)
    zf = jnp.zeros((16,), jnp.float32)
    negbig = jnp.full((16,), -_FLT_BIG, jnp.float32)
    posbig = jnp.full((16,), _FLT_BIG, jnp.float32)

    def init_acc(r, carry):
        for cg in range(8):
            sl = pl.ds(cg * 16, 16)
            accsum[r, sl] = zf
            accmax[r, sl] = negbig
            accmin[r, sl] = posbig
        return carry
    lax.fori_loop(0, _NN, init_acc, 0)

    def init_small(i, carry):
        degbuf[pl.ds(i * 16, 16)] = zf
        return carry
    lax.fori_loop(0, _NN // 16, init_small, 0)

    def init_stage(i, carry):
        sl = pl.ds(i * 16, 16)
        sstage[sl] = jnp.zeros((16,), jnp.int32)
        dstage[sl] = jnp.zeros((16,), jnp.int32)
        return carry
    lax.fori_loop(0, (_CH + 16) // 16, init_stage, 0)

    def chunk_body(c, carry):
        pltpu.sync_copy(dst_hbm.at[pl.ds(c * _CH, _CH)], dstbuf)
        pltpu.sync_copy(src_hbm.at[pl.ds(c * _CH, _CH)], srcbuf)

        def filt(g, ptr):
            dstv = dstbuf[pl.ds(g * 16, 16)]
            srcv = srcbuf[pl.ds(g * 16, 16)]
            m = (dstv >= lo) & (dstv < lo + _NN)
            pref = plsc.cumsum(m.astype(jnp.int32))
            pos = ptr + pref - 1
            plsc.store_scatter(sstage, [pos], srcv, mask=m)
            plsc.store_scatter(dstage, [pos], dstv - lo, mask=m)
            return ptr + jnp.max(pref)
        cnt = lax.fori_loop(0, _CH // 16, filt, jnp.int32(0))

        nb = (cnt + (_GB - 1)) // _GB

        def batch_body(b, carry2):
            base = b * _GB
            idxv = plsc.load_gather(sstage, [jnp.broadcast_to(base, (16,)) + iota])
            idxbuf[pl.ds(0, 16)] = idxv
            pltpu.async_copy(h_hbm.at[idxbuf], msgbuf, sem).wait()
            rem = cnt - base
            ones1 = jnp.ones((16,), jnp.float32)
            m0 = iota == 0
            for l in range(_GB):
                @pl.when(rem > l)
                def _edge():
                    dsp = plsc.load_gather(
                        dstage, [jnp.broadcast_to(base + l, (16,))])
                    lsp = jnp.full((16,), l, jnp.int32)
                    for cg in range(8):
                        col = iota + (cg * 16)
                        mv = plsc.load_gather(msgbuf, [lsp, col])
                        omx = plsc.load_gather(accmax, [dsp, col])
                        plsc.store_scatter(accmax, [dsp, col],
                                           jnp.maximum(omx, mv))
                        omn = plsc.load_gather(accmin, [dsp, col])
                        plsc.store_scatter(accmin, [dsp, col],
                                           jnp.minimum(omn, mv))
                        plsc.addupdate_scatter(accsum, [dsp, col], mv)
                    plsc.addupdate_scatter(degbuf, [dsp], ones1, mask=m0)
            return carry2
        lax.fori_loop(0, nb, batch_body, 0)
        return carry
    lax.fori_loop(0, _NCHUNK, chunk_body, 0)

    pltpu.sync_copy(accsum, osum.at[pl.ds(lo, _NN)])
    pltpu.sync_copy(accmax, omax.at[pl.ds(lo, _NN)])
    pltpu.sync_copy(accmin, omin.at[pl.ds(lo, _NN)])
    pltpu.sync_copy(degbuf, odeg.at[wid])


def _make_sc_agg():
    mesh = plsc.VectorSubcoreMesh(core_axis_name="c", subcore_axis_name="s")
    f32 = jnp.float32
    return pl.kernel(
        _sc_agg_body,
        out_type=(
            jax.ShapeDtypeStruct((_NPAD, _D), f32),
            jax.ShapeDtypeStruct((_NPAD, _D), f32),
            jax.ShapeDtypeStruct((_NPAD, _D), f32),
            jax.ShapeDtypeStruct((_NW, _NN), f32),
        ),
        mesh=mesh,
        compiler_params=pltpu.CompilerParams(needs_layout_passes=False),
        scratch_types=[
            pltpu.VMEM((_CH,), jnp.int32),
            pltpu.VMEM((_CH,), jnp.int32),
            pltpu.VMEM((_CH + 16,), jnp.int32),
            pltpu.VMEM((_CH + 16,), jnp.int32),
            pltpu.VMEM((16,), jnp.int32),
            pltpu.VMEM((_GB, _D), f32),
            pltpu.VMEM((_NN, _D), f32),
            pltpu.VMEM((_NN, _D), f32),
            pltpu.VMEM((_NN, _D), f32),
            pltpu.VMEM((_NN,), f32),
            pltpu.SemaphoreType.DMA,
        ],
    )


_BLK = 1000
_GRID = _N // _BLK


def _dense_body(hsum_ref, hmax_ref, hmin_ref, deg_ref, snorm_ref, w_ref, b_ref,
                out_ref, ps_ref, pq_ref):
    i = pl.program_id(0)
    deg = deg_ref[...]
    pos = deg > 0.0
    mean = hsum_ref[...] / jnp.maximum(deg, 1.0)
    mx = jnp.where(pos, hmax_ref[...], 0.0)
    mn = jnp.where(pos, hmin_ref[...], 0.0)
    hc = jnp.concatenate([mean, mx, mn], axis=1)
    ld = jnp.log(deg + 1.0)
    amp = ld * (1.0 / _AVG_D_LOG)
    att = _AVG_D_LOG / jnp.maximum(ld, 1e-6)
    t0 = jnp.dot(hc, w_ref[0:384, :], preferred_element_type=jnp.float32,
                 precision=lax.Precision.HIGHEST)
    t1 = jnp.dot(hc, w_ref[384:768, :], preferred_element_type=jnp.float32,
                 precision=lax.Precision.HIGHEST)
    t2 = jnp.dot(hc, w_ref[768:1152, :], preferred_element_type=jnp.float32,
                 precision=lax.Precision.HIGHEST)
    o = (t0 + amp * t1 + att * t2 + b_ref[...]) * snorm_ref[...]
    out_ref[...] = o

    @pl.when(i == 0)
    def _init():
        ps_ref[...] = jnp.zeros_like(ps_ref)
        pq_ref[...] = jnp.zeros_like(pq_ref)
    ps_ref[...] += jnp.sum(o, axis=0, keepdims=True)
    pq_ref[...] += jnp.sum(o * o, axis=0, keepdims=True)


def _bn_body(out1_ref, h_ref, ps_ref, pq_ref, gamma_ref, beta_ref, o_ref):
    inv_n = 1.0 / _N
    mu = ps_ref[...] * inv_n
    var = pq_ref[...] * inv_n - mu * mu
    inv = lax.rsqrt(var + _BN_EPS)
    o = (out1_ref[...] - mu) * inv * gamma_ref[...] + beta_ref[...]
    o_ref[...] = h_ref[...] + jnp.maximum(o, 0.0)


def kernel(h, edge_index, e, snorm_n, W, b, gamma, beta):
    del e
    ei = edge_index.astype(jnp.int32)
    src = ei[0]
    dst = ei[1]

    hsum_p, hmax_p, hmin_p, deg2 = _make_sc_agg()(src, dst, h)
    hsum = hsum_p[:_N]
    hmax = hmax_p[:_N]
    hmin = hmin_p[:_N]
    deg = deg2.reshape(-1)[:_N][:, None]

    f32 = jnp.float32
    row_spec = pl.BlockSpec((_BLK, _D), lambda i: (i, 0))
    col_spec = pl.BlockSpec((_BLK, 1), lambda i: (i, 0))
    full_spec = pl.BlockSpec((1152, _D), lambda i: (0, 0))
    vec_spec = pl.BlockSpec((1, _D), lambda i: (0, 0))

    out1, ps, pq = pl.pallas_call(
        _dense_body,
        grid=(_GRID,),
        in_specs=[row_spec, row_spec, row_spec, col_spec, col_spec,
                  full_spec, vec_spec],
        out_specs=[row_spec, vec_spec, vec_spec],
        out_shape=[
            jax.ShapeDtypeStruct((_N, _D), f32),
            jax.ShapeDtypeStruct((1, _D), f32),
            jax.ShapeDtypeStruct((1, _D), f32),
        ],
    )(hsum, hmax, hmin, deg, snorm_n, W, b.reshape(1, _D))

    out = pl.pallas_call(
        _bn_body,
        grid=(_GRID,),
        in_specs=[row_spec, row_spec, vec_spec, vec_spec, vec_spec, vec_spec],
        out_specs=row_spec,
        out_shape=jax.ShapeDtypeStruct((_N, _D), f32),
    )(out1, h, ps, pq, gamma.reshape(1, _D), beta.reshape(1, _D))
    return out
